# baseline (speedup 1.0000x reference)
import jax
import jax.numpy as jnp
from jax.experimental import pallas as pl
from jax.experimental.pallas import tpu as pltpu

_BN = 2048


def _mm_body(feat_ref, tfeat_ref, mem_ref, params_ref, out_t_ref, out_f_ref):
    t = params_ref[0, 0]
    inv_t = 1.0 / t
    m = mem_ref[...]
    dims = (((1,), (1,)), ((), ()))
    out_f_ref[...] = jax.lax.dot_general(
        m, feat_ref[...], dims, preferred_element_type=jnp.float32) * inv_t
    out_t_ref[...] = jax.lax.dot_general(
        m, tfeat_ref[...], dims,
        preferred_element_type=jnp.float32) * (inv_t * inv_t)


def _sim_body(feat_ref, tfeat_ref, sim_ref):
    sim_ref[...] = jnp.sum(feat_ref[...] * tfeat_ref[...], axis=-1,
                           keepdims=True)


def kernel(image_features, transformed_image_features, indices, memory, params):
    del indices
    B, D = image_features.shape
    N = memory.shape[0]
    p2d = params.reshape(1, 2)
    out_t, out_f = pl.pallas_call(
        _mm_body,
        grid=(pl.cdiv(N, _BN),),
        in_specs=[
            pl.BlockSpec((B, D), lambda j: (0, 0)),
            pl.BlockSpec((B, D), lambda j: (0, 0)),
            pl.BlockSpec((_BN, D), lambda j: (j, 0)),
            pl.BlockSpec((1, 2), lambda j: (0, 0)),
        ],
        out_specs=[
            pl.BlockSpec((_BN, B), lambda j: (j, 0)),
            pl.BlockSpec((_BN, B), lambda j: (j, 0)),
        ],
        out_shape=[
            jax.ShapeDtypeStruct((N, B), jnp.float32),
            jax.ShapeDtypeStruct((N, B), jnp.float32),
        ],
        compiler_params=pltpu.CompilerParams(
            dimension_semantics=("parallel",),
        ),
    )(image_features, transformed_image_features, memory, p2d)
    sim = pl.pallas_call(
        _sim_body,
        out_shape=jax.ShapeDtypeStruct((B, 1), jnp.float32),
    )(image_features, transformed_image_features)
    return (out_t.T, out_f.T, sim)


# speedup vs baseline: 1.0949x; 1.0949x over previous
import jax
import jax.numpy as jnp
from jax.experimental import pallas as pl
from jax.experimental.pallas import tpu as pltpu

_BN = 2048


def _body(feat_ref, tfeat_ref, memt_ref, params_ref, out_t_ref, out_f_ref,
          sim_ref):
    j = pl.program_id(0)
    t = params_ref[0, 0]
    inv_t = 1.0 / t
    f = feat_ref[...]
    tf = tfeat_ref[...]
    mt = memt_ref[:, pl.ds(j * _BN, _BN)]
    dims = (((0,), (1,)), ((), ()))
    out_f_ref[...] = jax.lax.dot_general(
        mt, f, dims, preferred_element_type=jnp.float32) * inv_t
    out_t_ref[...] = jax.lax.dot_general(
        mt, tf, dims, preferred_element_type=jnp.float32) * (inv_t * inv_t)

    @pl.when(j == 0)
    def _():
        sim_ref[...] = jnp.sum(f * tf, axis=-1, keepdims=True)


def kernel(image_features, transformed_image_features, indices, memory, params):
    del indices
    B, D = image_features.shape
    N = memory.shape[0]
    nb = pl.cdiv(N, _BN)
    n_pad = nb * _BN
    memt = jnp.pad(memory.T, ((0, 0), (0, n_pad - N)))
    p2d = params.reshape(1, 2)
    out_t, out_f, sim = pl.pallas_call(
        _body,
        grid=(nb,),
        in_specs=[
            pl.BlockSpec((B, D), lambda j: (0, 0)),
            pl.BlockSpec((B, D), lambda j: (0, 0)),
            pl.BlockSpec((D, n_pad), lambda j: (0, 0)),
            pl.BlockSpec((1, 2), lambda j: (0, 0)),
        ],
        out_specs=[
            pl.BlockSpec((_BN, B), lambda j: (j, 0)),
            pl.BlockSpec((_BN, B), lambda j: (j, 0)),
            pl.BlockSpec((B, 1), lambda j: (0, 0)),
        ],
        out_shape=[
            jax.ShapeDtypeStruct((N, B), jnp.float32),
            jax.ShapeDtypeStruct((N, B), jnp.float32),
            jax.ShapeDtypeStruct((B, 1), jnp.float32),
        ],
        compiler_params=pltpu.CompilerParams(
            dimension_semantics=("arbitrary",),
            vmem_limit_bytes=64 * 1024 * 1024,
        ),
    )(image_features, transformed_image_features, memt, p2d)
    return (out_t.T, out_f.T, sim)
